# baseline (speedup 1.0000x reference)
import jax
import jax.numpy as jnp
from jax import lax
from jax.experimental import pallas as pl
from jax.experimental.pallas import tpu as pltpu
from jax.experimental.pallas import tpu_sc as plsc

N_USER = 1000000
N_ITEM = 1000000
DIM = 64
BATCH = 16384

NC = 2
NS = 16
NW = NC * NS
B_PER_W = BATCH // NW
IDX_CHUNK = 128
N_CHUNK = B_PER_W // IDX_CHUNK
GROUPS = IDX_CHUNK // 16


def _mf_body(u_hbm, i_hbm, U_hbm, V_hbm, out_hbm,
             u_idx, i_idx, pu, pi, u_rows, v_rows, out_v, sem):
    wid = lax.axis_index("s") * NC + lax.axis_index("c")

    pltpu.sync_copy(u_hbm.at[pl.ds(wid * N_CHUNK, N_CHUNK)], u_idx)
    pltpu.sync_copy(i_hbm.at[pl.ds(wid * N_CHUNK, N_CHUNK)], i_idx)

    iota16 = lax.iota(jnp.int32, 16)

    for j in range(N_CHUNK):
        def shift(k, carry):
            s = pl.ds(k * 16, 16)
            pu[s] = jnp.right_shift(u_idx[j, s], 1)
            pi[s] = jnp.right_shift(i_idx[j, s], 1)
            return carry
        lax.fori_loop(0, GROUPS, shift, 0, unroll=True)

        cu = pltpu.make_async_copy(U_hbm.at[pu], u_rows, sem)
        cv = pltpu.make_async_copy(V_hbm.at[pi], v_rows, sem)
        cu.start()
        cv.start()
        cu.wait()
        cv.wait()

        def group(g, carry):
            s = pl.ds(g * 16, 16)
            rows = g * 16 + iota16
            ucol = jnp.left_shift(jnp.bitwise_and(u_idx[j, s], 1), 6)
            vcol = jnp.left_shift(jnp.bitwise_and(i_idx[j, s], 1), 6)
            acc = jnp.zeros((16,), jnp.float32)
            for d in range(DIM):
                uu = plsc.load_gather(u_rows, [rows, ucol + d])
                vv = plsc.load_gather(v_rows, [rows, vcol + d])
                acc = acc + uu * vv
            out_v[pl.ds(j * IDX_CHUNK + g * 16, 16)] = acc
            return carry

        lax.fori_loop(0, GROUPS, group, 0, unroll=False)

    pltpu.sync_copy(out_v, out_hbm.at[pl.ds(wid * B_PER_W, B_PER_W)])


@jax.jit
def kernel(u, i, U_emb, V_emb):
    u2 = u.reshape(NW * N_CHUNK, IDX_CHUNK)
    i2 = i.reshape(NW * N_CHUNK, IDX_CHUNK)
    U2 = U_emb.reshape(N_USER // 2, 2 * DIM)
    V2 = V_emb.reshape(N_ITEM // 2, 2 * DIM)
    mesh = plsc.VectorSubcoreMesh(core_axis_name="c", subcore_axis_name="s")
    f = pl.kernel(
        _mf_body,
        out_type=jax.ShapeDtypeStruct((BATCH,), jnp.float32),
        mesh=mesh,
        compiler_params=pltpu.CompilerParams(needs_layout_passes=False),
        scratch_types=[
            pltpu.VMEM((N_CHUNK, IDX_CHUNK), jnp.int32),
            pltpu.VMEM((N_CHUNK, IDX_CHUNK), jnp.int32),
            pltpu.VMEM((IDX_CHUNK,), jnp.int32),
            pltpu.VMEM((IDX_CHUNK,), jnp.int32),
            pltpu.VMEM((IDX_CHUNK, 2 * DIM), jnp.float32),
            pltpu.VMEM((IDX_CHUNK, 2 * DIM), jnp.float32),
            pltpu.VMEM((B_PER_W,), jnp.float32),
            pltpu.SemaphoreType.DMA,
        ],
    )
    return f(u2, i2, U2, V2)


# speedup vs baseline: 1.0006x; 1.0006x over previous
import jax
import jax.numpy as jnp
from jax import lax
from jax.experimental import pallas as pl
from jax.experimental.pallas import tpu as pltpu
from jax.experimental.pallas import tpu_sc as plsc

N_USER = 1000000
N_ITEM = 1000000
DIM = 64
BATCH = 16384

NC = 2
NS = 16
NW = NC * NS
B_PER_W = BATCH // NW
IDX_CHUNK = 128
N_CHUNK = B_PER_W // IDX_CHUNK
GROUPS = IDX_CHUNK // 16


def _mf_body(u_hbm, i_hbm, U_hbm, V_hbm, out_hbm,
             u_idx, i_idx, pu, pi, u_rows, v_rows, out_v, sem):
    wid = lax.axis_index("s") * NC + lax.axis_index("c")

    pltpu.sync_copy(u_hbm.at[pl.ds(wid * N_CHUNK, N_CHUNK)], u_idx)
    pltpu.sync_copy(i_hbm.at[pl.ds(wid * N_CHUNK, N_CHUNK)], i_idx)

    iota16 = lax.iota(jnp.int32, 16)

    for j in range(N_CHUNK):
        def shift(k, carry):
            s = pl.ds(k * 16, 16)
            pu[s] = jnp.right_shift(u_idx[j, s], 1)
            pi[s] = jnp.right_shift(i_idx[j, s], 1)
            return carry
        lax.fori_loop(0, GROUPS, shift, 0, unroll=True)

        cu = pltpu.make_async_copy(U_hbm.at[pu], u_rows, sem)
        cv = pltpu.make_async_copy(V_hbm.at[pi], v_rows, sem)
        cu.start()
        cv.start()
        cu.wait()
        cv.wait()

        def group(g, carry):
            s = pl.ds(g * 16, 16)
            rows = g * 16 + iota16
            ucol = jnp.left_shift(jnp.bitwise_and(u_idx[j, s], 1), 6)
            vcol = jnp.left_shift(jnp.bitwise_and(i_idx[j, s], 1), 6)
            acc = jnp.zeros((16,), jnp.float32)
            for d in range(DIM):
                uu = plsc.load_gather(u_rows, [rows, ucol + d])
                vv = plsc.load_gather(v_rows, [rows, vcol + d])
                acc = acc + uu * vv
            out_v[pl.ds(j * IDX_CHUNK + g * 16, 16)] = acc
            return carry

        lax.fori_loop(0, GROUPS, group, 0, unroll=False)

    pltpu.sync_copy(out_v, out_hbm.at[pl.ds(wid * B_PER_W, B_PER_W)])


@jax.jit
def kernel(u, i, U_emb, V_emb):
    u2 = u.reshape(NW * N_CHUNK, IDX_CHUNK)
    i2 = i.reshape(NW * N_CHUNK, IDX_CHUNK)
    U2 = U_emb.reshape(N_USER // 2, 2 * DIM)
    V2 = V_emb.reshape(N_ITEM // 2, 2 * DIM)
    mesh = plsc.VectorSubcoreMesh(core_axis_name="c", subcore_axis_name="s")
    f = pl.kernel(
        _mf_body,
        out_type=jax.ShapeDtypeStruct((BATCH,), jnp.float32),
        mesh=mesh,
        compiler_params=pltpu.CompilerParams(
            needs_layout_passes=False, use_tc_tiling_on_sc=True),
        scratch_types=[
            pltpu.VMEM((N_CHUNK, IDX_CHUNK), jnp.int32),
            pltpu.VMEM((N_CHUNK, IDX_CHUNK), jnp.int32),
            pltpu.VMEM((IDX_CHUNK,), jnp.int32),
            pltpu.VMEM((IDX_CHUNK,), jnp.int32),
            pltpu.VMEM((IDX_CHUNK, 2 * DIM), jnp.float32),
            pltpu.VMEM((IDX_CHUNK, 2 * DIM), jnp.float32),
            pltpu.VMEM((B_PER_W,), jnp.float32),
            pltpu.SemaphoreType.DMA,
        ],
    )
    return f(u2, i2, U2, V2)
